# Initial kernel scaffold; baseline (speedup 1.0000x reference)
import functools

import jax
import jax.numpy as jnp
from jax import lax
from jax.experimental import pallas as pl
from jax.experimental.pallas import tpu as pltpu
from jax.experimental.pallas import tpu_sc as plsc

N = 50000
E = 800000
DIN = 64
DH = 64
DE = 32
B = 16384

NC = 2
NS = 16
R = 3128
N_PAD = NS * R
K = 1000

_mesh = plsc.VectorSubcoreMesh(core_axis_name="c", subcore_axis_name="s")


def _zero_rows(rows_v):
    def body(i, _):
        r = i // 2
        off = (i % 2) * 16
        rows_v[r, pl.ds(off, 16)] = jnp.zeros((16,), jnp.float32)
        return 0
    lax.fori_loop(0, K * DE // 16, body, 0)


def _fill_1d(ref, n, value):
    def body(i, _):
        ref[pl.ds(i * 16, 16)] = jnp.full((16,), value, jnp.float32)
        return 0
    lax.fori_loop(0, n // 16, body, 0)


def _make_scatter(two_tables: bool, do_deg: bool):
    n_chunks = (E // NS // K) if two_tables else (E // NC // NS // K)

    out_types = [jax.ShapeDtypeStruct((NC * N_PAD, DE), jnp.float32)]
    if do_deg:
        out_types.append(jax.ShapeDtypeStruct((NC * N_PAD,), jnp.float32))

    scratch = [
        pltpu.VMEM_SHARED((N_PAD, DE), jnp.float32),
        pltpu.VMEM((K,), jnp.int32),
        pltpu.VMEM((K,), jnp.int32),
        pltpu.VMEM((K, DE), jnp.float32),
        pltpu.SemaphoreType.DMA,
    ]
    if do_deg:
        scratch += [
            pltpu.VMEM_SHARED((N_PAD,), jnp.float32),
            pltpu.VMEM((1024,), jnp.float32),
            pltpu.VMEM((1024,), jnp.float32),
        ]

    def body(*refs):
        if two_tables:
            t0, t1, src_hbm, dst_hbm = refs[:4]
            refs = refs[4:]
        else:
            t0, src_hbm, dst_hbm = refs[:3]
            t1 = t0
            refs = refs[3:]
        if do_deg:
            out_hbm, degp_hbm = refs[0], refs[1]
            acc, src_v, dst_v, rows_v, sem, deg_acc, ones_v, zflat_v = refs[2:]
        else:
            out_hbm = refs[0]
            acc, src_v, dst_v, rows_v, sem = refs[1:]

        c = lax.axis_index("c")
        s = lax.axis_index("s")

        _zero_rows(rows_v)
        base_r = s * R
        for j in range(3):
            pltpu.sync_copy(rows_v, acc.at[pl.ds(base_r + j * K, K)])
        pltpu.sync_copy(rows_v.at[pl.ds(0, R - 3 * K)],
                        acc.at[pl.ds(base_r + 3 * K, R - 3 * K)])
        if do_deg:
            _fill_1d(zflat_v, 1024, 0.0)
            _fill_1d(ones_v, 1024, 1.0)
            for j in range(3):
                pltpu.sync_copy(zflat_v, deg_acc.at[pl.ds(base_r + j * 1024, 1024)])
            pltpu.sync_copy(zflat_v.at[pl.ds(0, R - 3 * 1024)],
                            deg_acc.at[pl.ds(base_r + 3 * 1024, R - 3 * 1024)])
        plsc.subcore_barrier()

        if two_tables:
            base_t = s * (E // NS)
        else:
            base_t = c * (E // NC) + s * (E // NC // NS)

        def chunk(i, table_ref):
            base = base_t + i * K
            pltpu.sync_copy(src_hbm.at[pl.ds(base, K)], src_v)
            pltpu.sync_copy(dst_hbm.at[pl.ds(base, K)], dst_v)
            pltpu.async_copy(table_ref.at[src_v], rows_v, sem).wait()
            pltpu.sync_copy(rows_v, acc.at[dst_v], add=True)
            if do_deg:
                @pl.when((s // (NS // NC)) == c)
                def _():
                    pltpu.sync_copy(ones_v.at[pl.ds(0, K)],
                                    deg_acc.at[dst_v], add=True)

        if two_tables:
            @pl.when(c == 0)
            def _():
                lax.fori_loop(0, n_chunks, lambda i, _: (chunk(i, t0), 0)[1], 0)

            @pl.when(c == 1)
            def _():
                lax.fori_loop(0, n_chunks, lambda i, _: (chunk(i, t1), 0)[1], 0)
        else:
            lax.fori_loop(0, n_chunks, lambda i, _: (chunk(i, t0), 0)[1], 0)

        plsc.subcore_barrier()

        out_base = c * N_PAD + base_r
        pltpu.sync_copy(acc.at[pl.ds(base_r, R)], out_hbm.at[pl.ds(out_base, R)])
        if do_deg:
            pltpu.sync_copy(deg_acc.at[pl.ds(base_r, R)],
                            degp_hbm.at[pl.ds(out_base, R)])

    return pl.kernel(body, out_type=tuple(out_types), mesh=_mesh,
                     scratch_types=scratch)


_scatter1 = _make_scatter(two_tables=True, do_deg=True)
_scatter2 = _make_scatter(two_tables=False, do_deg=False)


B3 = B * 3
_TG = B3 // (NC * NS)


def _team_gather_body(emb_hbm, tidx_hbm, trows_hbm, idx_v, rows_v, sem):
    c = lax.axis_index("c")
    s = lax.axis_index("s")
    w = s * NC + c
    base = w * _TG
    pltpu.sync_copy(tidx_hbm.at[pl.ds(base, _TG)], idx_v)
    pltpu.async_copy(emb_hbm.at[idx_v], rows_v, sem).wait()
    pltpu.sync_copy(rows_v, trows_hbm.at[pl.ds(base, _TG)])


_team_gather = pl.kernel(
    _team_gather_body,
    out_type=jax.ShapeDtypeStruct((B3, DE), jnp.float32),
    mesh=_mesh,
    scratch_types=[
        pltpu.VMEM((_TG,), jnp.int32),
        pltpu.VMEM((_TG, DE), jnp.float32),
        pltpu.SemaphoreType.DMA,
    ],
)


BN = 1000
GRID_N = N // BN


def _prep_body(x_ref, w1lt_ref, w1rt_ref, b1_ref, t0_ref, t1_ref, q_ref):
    xb = x_ref[...]
    t = jnp.dot(xb, w1lt_ref[...], preferred_element_type=jnp.float32)
    t0_ref[...] = t[:, :DE]
    t1_ref[...] = t[:, DE:]
    q_ref[...] = jnp.dot(xb, w1rt_ref[...],
                         preferred_element_type=jnp.float32) + b1_ref[...]


def _prep(x, w1lt, w1rt, b1row):
    return pl.pallas_call(
        _prep_body,
        grid=(GRID_N,),
        in_specs=[
            pl.BlockSpec((BN, DIN), lambda i: (i, 0)),
            pl.BlockSpec((DIN, DH), lambda i: (0, 0)),
            pl.BlockSpec((DIN, DH), lambda i: (0, 0)),
            pl.BlockSpec((1, DH), lambda i: (0, 0)),
        ],
        out_specs=[
            pl.BlockSpec((BN, DE), lambda i: (i, 0)),
            pl.BlockSpec((BN, DE), lambda i: (i, 0)),
            pl.BlockSpec((BN, DH), lambda i: (i, 0)),
        ],
        out_shape=[
            jax.ShapeDtypeStruct((N, DE), jnp.float32),
            jax.ShapeDtypeStruct((N, DE), jnp.float32),
            jax.ShapeDtypeStruct((N, DH), jnp.float32),
        ],
    )(x, w1lt, w1rt, b1row)


def _mid_body(s1_ref, degp_ref, q_ref, w2lt_ref, w2rt_ref, b2_ref,
              m2_ref, r2_ref, recip_ref):
    deg = degp_ref[0] + degp_ref[1]
    rc = 1.0 / jnp.maximum(deg, 1.0)
    mean = jnp.concatenate([s1_ref[0], s1_ref[1]], axis=1) * rc[:, None]
    h = jnp.maximum(mean + q_ref[...], 0.0)
    m2_ref[...] = jnp.dot(h, w2lt_ref[...], preferred_element_type=jnp.float32)
    r2_ref[...] = jnp.dot(h, w2rt_ref[...],
                          preferred_element_type=jnp.float32) + b2_ref[...]
    recip_ref[...] = rc


def _mid(s1, degp, q, w2lt, w2rt, b2row):
    return pl.pallas_call(
        _mid_body,
        grid=(GRID_N,),
        in_specs=[
            pl.BlockSpec((NC, BN, DE), lambda i: (0, i, 0)),
            pl.BlockSpec((NC, BN), lambda i: (0, i)),
            pl.BlockSpec((BN, DH), lambda i: (i, 0)),
            pl.BlockSpec((DH, DE), lambda i: (0, 0)),
            pl.BlockSpec((DH, DE), lambda i: (0, 0)),
            pl.BlockSpec((1, DE), lambda i: (0, 0)),
        ],
        out_specs=[
            pl.BlockSpec((BN, DE), lambda i: (i, 0)),
            pl.BlockSpec((BN, DE), lambda i: (i, 0)),
            pl.BlockSpec((BN,), lambda i: (i,)),
        ],
        out_shape=[
            jax.ShapeDtypeStruct((N, DE), jnp.float32),
            jax.ShapeDtypeStruct((N, DE), jnp.float32),
            jax.ShapeDtypeStruct((N,), jnp.float32),
        ],
    )(s1, degp, q, w2lt, w2rt, b2row)


def _post_body(s2_ref, recip_ref, r2_ref, emb_ref):
    emb_ref[...] = ((s2_ref[0] + s2_ref[1]) * recip_ref[...][:, None]
                    + r2_ref[...])


def _post(s2, recip, r2):
    return pl.pallas_call(
        _post_body,
        grid=(GRID_N,),
        in_specs=[
            pl.BlockSpec((NC, BN, DE), lambda i: (0, i, 0)),
            pl.BlockSpec((BN,), lambda i: (i,)),
            pl.BlockSpec((BN, DE), lambda i: (i, 0)),
        ],
        out_specs=pl.BlockSpec((BN, DE), lambda i: (i, 0)),
        out_shape=jax.ShapeDtypeStruct((N, DE), jnp.float32),
    )(s2, recip, r2)


BD = 1024


def _dur_body(tvec_ref, wpt_ref, bp_ref, out_ref):
    out_ref[...] = jnp.dot(tvec_ref[...], wpt_ref[...],
                           preferred_element_type=jnp.float32) + bp_ref[...]


def _dur(tvec, wpt, bp11):
    return pl.pallas_call(
        _dur_body,
        grid=(B // BD,),
        in_specs=[
            pl.BlockSpec((BD, 3 * DE), lambda i: (i, 0)),
            pl.BlockSpec((3 * DE, 1), lambda i: (0, 0)),
            pl.BlockSpec((1, 1), lambda i: (0, 0)),
        ],
        out_specs=pl.BlockSpec((BD, 1), lambda i: (i, 0)),
        out_shape=jax.ShapeDtypeStruct((B, 1), jnp.float32),
    )(tvec, wpt, bp11)


def kernel(x, edge_index, team_indices, W1l, b1, W1r, W2l, b2, W2r, Wp, bp):
    src = edge_index[0]
    dst = edge_index[1]

    t0, t1, q = _prep(x, W1l.T, W1r.T, b1.reshape(1, DH))
    s1_flat, degp_flat = _scatter1(t0, t1, src, dst)
    s1 = s1_flat.reshape(NC, N_PAD, DE)
    degp = degp_flat.reshape(NC, N_PAD)

    m2, r2, recip = _mid(s1, degp, q, W2l.T, W2r.T, b2.reshape(1, DE))
    (s2_flat,) = _scatter2(m2, src, dst)
    s2 = s2_flat.reshape(NC, N_PAD, DE)

    emb = _post(s2, recip, r2)

    trows = _team_gather(emb, team_indices.reshape(B3))
    dur = _dur(trows.reshape(B, 3 * DE), Wp.T, bp.reshape(1, 1))
    return emb, dur[:, 0]


# speedup vs baseline: 8.7707x; 8.7707x over previous
import functools

import jax
import jax.numpy as jnp
from jax import lax
from jax.experimental import pallas as pl
from jax.experimental.pallas import tpu as pltpu
from jax.experimental.pallas import tpu_sc as plsc

N = 50000
E = 800000
DIN = 64
DH = 64
DE = 32
B = 16384

NC = 2
NS = 16
R = 3128
N_PAD = NS * R
K1 = 400
K2 = 1000
DE2 = DE // 2

@functools.cache
def _get_mesh():
    return plsc.VectorSubcoreMesh(core_axis_name="c", subcore_axis_name="s",
                                  num_cores=NC, num_subcores=NS)


def _fill_1d(ref, n, value):
    def body(i, _):
        ref[pl.ds(i * 16, 16)] = jnp.full((16,), value, jnp.float32)
        return 0
    lax.fori_loop(0, n // 16, body, 0)


def _make_scatter(width: int, k: int, do_deg: bool):
    n_chunks = E // NS // k

    out_types = [jax.ShapeDtypeStruct((NC * N_PAD, width), jnp.float32)]
    if do_deg:
        out_types.append(jax.ShapeDtypeStruct((NC * N_PAD,), jnp.float32))

    scratch = [
        pltpu.VMEM_SHARED((N_PAD, width), jnp.float32),
        pltpu.VMEM((k,), jnp.int32),
        pltpu.VMEM((k,), jnp.int32),
        pltpu.VMEM((k, width), jnp.float32),
        pltpu.SemaphoreType.DMA,
    ]
    if do_deg:
        scratch += [
            pltpu.VMEM_SHARED((N_PAD,), jnp.float32),
            pltpu.VMEM((1024,), jnp.float32),
            pltpu.VMEM((1024,), jnp.float32),
        ]

    def body(*refs):
        t0, t1, src_hbm, dst_hbm = refs[:4]
        refs = refs[4:]
        if do_deg:
            out_hbm, degp_hbm = refs[0], refs[1]
            acc, src_v, dst_v, rows_v, sem, deg_acc, ones_v, zflat_v = refs[2:]
        else:
            out_hbm = refs[0]
            acc, src_v, dst_v, rows_v, sem = refs[1:]

        c = lax.axis_index("c")
        s = lax.axis_index("s")

        def zero_rows(i, _):
            rw = i // (width // 16)
            off = (i % (width // 16)) * 16
            rows_v[rw, pl.ds(off, 16)] = jnp.zeros((16,), jnp.float32)
            return 0
        lax.fori_loop(0, k * width // 16, zero_rows, 0)

        base_r = s * R
        nfull, rem = R // k, R % k
        for j in range(nfull):
            pltpu.sync_copy(rows_v, acc.at[pl.ds(base_r + j * k, k)])
        if rem:
            pltpu.sync_copy(rows_v.at[pl.ds(0, rem)],
                            acc.at[pl.ds(base_r + nfull * k, rem)])
        if do_deg:
            _fill_1d(zflat_v, 1024, 0.0)
            _fill_1d(ones_v, 1024, 1.0)
            for j in range(3):
                pltpu.sync_copy(zflat_v, deg_acc.at[pl.ds(base_r + j * 1024, 1024)])
            pltpu.sync_copy(zflat_v.at[pl.ds(0, R - 3 * 1024)],
                            deg_acc.at[pl.ds(base_r + 3 * 1024, R - 3 * 1024)])
        plsc.subcore_barrier()

        base_t = s * (E // NS)

        def chunk(i, table_ref):
            base = base_t + i * k
            pltpu.sync_copy(src_hbm.at[pl.ds(base, k)], src_v)
            pltpu.sync_copy(dst_hbm.at[pl.ds(base, k)], dst_v)
            pltpu.async_copy(table_ref.at[src_v], rows_v, sem).wait()
            pltpu.sync_copy(rows_v, acc.at[dst_v], add=True)
            if do_deg:
                @pl.when((s // (NS // NC)) == c)
                def _():
                    pltpu.sync_copy(ones_v.at[pl.ds(0, k)],
                                    deg_acc.at[dst_v], add=True)

        @pl.when(c == 0)
        def _():
            lax.fori_loop(0, n_chunks, lambda i, _: (chunk(i, t0), 0)[1], 0)

        @pl.when(c == 1)
        def _():
            lax.fori_loop(0, n_chunks, lambda i, _: (chunk(i, t1), 0)[1], 0)

        plsc.subcore_barrier()

        out_base = c * N_PAD + base_r
        pltpu.sync_copy(acc.at[pl.ds(base_r, R)], out_hbm.at[pl.ds(out_base, R)])
        if do_deg:
            pltpu.sync_copy(deg_acc.at[pl.ds(base_r, R)],
                            degp_hbm.at[pl.ds(out_base, R)])

    return pl.kernel(body, out_type=tuple(out_types), mesh=_get_mesh(),
                     scratch_types=scratch,
                     compiler_params=pltpu.CompilerParams(
                         use_tc_tiling_on_sc=False))


_scatter_cache = {}


def _get_scatter(width: int, k: int, do_deg: bool):
    key = (width, k, do_deg)
    if key not in _scatter_cache:
        _scatter_cache[key] = _make_scatter(width, k, do_deg)
    return _scatter_cache[key]


B3 = B * 3
_TG = B3 // (NC * NS)


def _team_gather_body(emb_hbm, tidx_hbm, trows_hbm, idx_v, rows_v, sem):
    c = lax.axis_index("c")
    s = lax.axis_index("s")
    w = s * NC + c
    base = w * _TG
    pltpu.sync_copy(tidx_hbm.at[pl.ds(base, _TG)], idx_v)
    pltpu.async_copy(emb_hbm.at[idx_v], rows_v, sem).wait()
    pltpu.sync_copy(rows_v, trows_hbm.at[pl.ds(base, _TG)])


@functools.cache
def _get_team_gather():
    return pl.kernel(
        _team_gather_body,
        out_type=jax.ShapeDtypeStruct((B3, DE), jnp.float32),
        mesh=_get_mesh(),
        scratch_types=[
            pltpu.VMEM((_TG,), jnp.int32),
            pltpu.VMEM((_TG, DE), jnp.float32),
            pltpu.SemaphoreType.DMA,
        ],
        compiler_params=pltpu.CompilerParams(use_tc_tiling_on_sc=False),
    )


BN = 1024
GRID_N = -(-N // BN)


def _prep_body(x_ref, w1lt_ref, w1rt_ref, b1_ref, t0_ref, t1_ref, q_ref):
    xb = x_ref[...]
    t = jnp.dot(xb, w1lt_ref[...], preferred_element_type=jnp.float32)
    t0_ref[...] = t[:, :DE]
    t1_ref[...] = t[:, DE:]
    q_ref[...] = jnp.dot(xb, w1rt_ref[...],
                         preferred_element_type=jnp.float32) + b1_ref[...]


def _prep(x, w1lt, w1rt, b1row):
    return pl.pallas_call(
        _prep_body,
        grid=(GRID_N,),
        in_specs=[
            pl.BlockSpec((BN, DIN), lambda i: (i, 0)),
            pl.BlockSpec((DIN, DH), lambda i: (0, 0)),
            pl.BlockSpec((DIN, DH), lambda i: (0, 0)),
            pl.BlockSpec((1, DH), lambda i: (0, 0)),
        ],
        out_specs=[
            pl.BlockSpec((BN, DE), lambda i: (i, 0)),
            pl.BlockSpec((BN, DE), lambda i: (i, 0)),
            pl.BlockSpec((BN, DH), lambda i: (i, 0)),
        ],
        out_shape=[
            jax.ShapeDtypeStruct((N, DE), jnp.float32),
            jax.ShapeDtypeStruct((N, DE), jnp.float32),
            jax.ShapeDtypeStruct((N, DH), jnp.float32),
        ],
    )(x, w1lt, w1rt, b1row)


def _mid_body(s1_ref, degp_ref, q_ref, w2lt_ref, w2rt_ref, b2_ref,
              m2a_ref, m2b_ref, r2_ref, recip_ref):
    deg = degp_ref[0] + degp_ref[1]
    rc = 1.0 / jnp.maximum(deg, 1.0)
    mean = jnp.concatenate([s1_ref[0], s1_ref[1]], axis=1) * rc[:, None]
    h = jnp.maximum(mean + q_ref[...], 0.0)
    m2 = jnp.dot(h, w2lt_ref[...], preferred_element_type=jnp.float32)
    m2a_ref[...] = m2[:, :DE2]
    m2b_ref[...] = m2[:, DE2:]
    r2_ref[...] = jnp.dot(h, w2rt_ref[...],
                          preferred_element_type=jnp.float32) + b2_ref[...]
    recip_ref[...] = rc


def _mid(s1, degp, q, w2lt, w2rt, b2row):
    return pl.pallas_call(
        _mid_body,
        grid=(GRID_N,),
        in_specs=[
            pl.BlockSpec((NC, BN, DE), lambda i: (0, i, 0)),
            pl.BlockSpec((NC, BN), lambda i: (0, i)),
            pl.BlockSpec((BN, DH), lambda i: (i, 0)),
            pl.BlockSpec((DH, DE), lambda i: (0, 0)),
            pl.BlockSpec((DH, DE), lambda i: (0, 0)),
            pl.BlockSpec((1, DE), lambda i: (0, 0)),
        ],
        out_specs=[
            pl.BlockSpec((BN, DE2), lambda i: (i, 0)),
            pl.BlockSpec((BN, DE2), lambda i: (i, 0)),
            pl.BlockSpec((BN, DE), lambda i: (i, 0)),
            pl.BlockSpec((BN,), lambda i: (i,)),
        ],
        out_shape=[
            jax.ShapeDtypeStruct((N, DE2), jnp.float32),
            jax.ShapeDtypeStruct((N, DE2), jnp.float32),
            jax.ShapeDtypeStruct((N, DE), jnp.float32),
            jax.ShapeDtypeStruct((N,), jnp.float32),
        ],
    )(s1, degp, q, w2lt, w2rt, b2row)


def _post_body(s2_ref, recip_ref, r2_ref, emb_ref):
    mean2 = jnp.concatenate([s2_ref[0], s2_ref[1]], axis=1)
    emb_ref[...] = mean2 * recip_ref[...][:, None] + r2_ref[...]


def _post(s2, recip, r2):
    return pl.pallas_call(
        _post_body,
        grid=(GRID_N,),
        in_specs=[
            pl.BlockSpec((NC, BN, DE2), lambda i: (0, i, 0)),
            pl.BlockSpec((BN,), lambda i: (i,)),
            pl.BlockSpec((BN, DE), lambda i: (i, 0)),
        ],
        out_specs=pl.BlockSpec((BN, DE), lambda i: (i, 0)),
        out_shape=jax.ShapeDtypeStruct((N, DE), jnp.float32),
    )(s2, recip, r2)


BD = 1024


def _dur_body(tvec_ref, wpt_ref, bp_ref, out_ref):
    out_ref[...] = jnp.dot(tvec_ref[...], wpt_ref[...],
                           preferred_element_type=jnp.float32) + bp_ref[...]


def _dur(tvec, wpt, bp11):
    return pl.pallas_call(
        _dur_body,
        grid=(B // BD,),
        in_specs=[
            pl.BlockSpec((BD, 3 * DE), lambda i: (i, 0)),
            pl.BlockSpec((3 * DE, 1), lambda i: (0, 0)),
            pl.BlockSpec((1, 1), lambda i: (0, 0)),
        ],
        out_specs=pl.BlockSpec((BD, 1), lambda i: (i, 0)),
        out_shape=jax.ShapeDtypeStruct((B, 1), jnp.float32),
    )(tvec, wpt, bp11)


def kernel(x, edge_index, team_indices, W1l, b1, W1r, W2l, b2, W2r, Wp, bp):
    src = edge_index[0]
    dst = edge_index[1]

    t0, t1, q = _prep(x, W1l.T, W1r.T, b1.reshape(1, DH))
    s1_flat, degp_flat = _get_scatter(DE, K1, True)(t0, t1, src, dst)
    s1 = s1_flat.reshape(NC, N_PAD, DE)
    degp = degp_flat.reshape(NC, N_PAD)

    m2a, m2b, r2, recip = _mid(s1, degp, q, W2l.T, W2r.T, b2.reshape(1, DE))
    (s2_flat,) = _get_scatter(DE2, K2, False)(m2a, m2b, src, dst)
    s2 = s2_flat.reshape(NC, N_PAD, DE2)

    emb = _post(s2, recip, r2)

    trows = _get_team_gather()(emb, team_indices.reshape(B3))
    dur = _dur(trows.reshape(B, 3 * DE), Wp.T, bp.reshape(1, 1))
    return emb, dur[:, 0]


# speedup vs baseline: 10.1836x; 1.1611x over previous
import functools

import jax
import jax.numpy as jnp
from jax import lax
from jax.experimental import pallas as pl
from jax.experimental.pallas import tpu as pltpu
from jax.experimental.pallas import tpu_sc as plsc

N = 50000
E = 800000
DIN = 64
DH = 64
DE = 32
B = 16384

NC = 2
NS = 16
R = 3128
N_PAD = NS * R
K2 = 1000
DE2 = DE // 2


@functools.cache
def _get_mesh():
    return plsc.VectorSubcoreMesh(core_axis_name="c", subcore_axis_name="s",
                                  num_cores=NC, num_subcores=NS)


def _fill_1d(ref, n, value):
    def body(i, _):
        ref[pl.ds(i * 16, 16)] = jnp.full((16,), value, jnp.float32)
        return 0
    lax.fori_loop(0, n // 16, body, 0)


def _make_scatter(n_tables: int, k: int, do_deg: bool):
    passes = n_tables // NC
    n = E // NS // k
    W = 16

    out_types = [jax.ShapeDtypeStruct((n_tables * N_PAD, W), jnp.float32)]
    if do_deg:
        out_types.append(jax.ShapeDtypeStruct((NC * N_PAD,), jnp.float32))

    scratch = [
        pltpu.VMEM_SHARED((N_PAD, W), jnp.float32),
        pltpu.VMEM((k,), jnp.int32), pltpu.VMEM((k,), jnp.int32),
        pltpu.VMEM((k,), jnp.int32), pltpu.VMEM((k,), jnp.int32),
        pltpu.VMEM((k, W), jnp.float32), pltpu.VMEM((k, W), jnp.float32),
        pltpu.SemaphoreType.DMA, pltpu.SemaphoreType.DMA,
        pltpu.SemaphoreType.DMA, pltpu.SemaphoreType.DMA,
    ]
    if do_deg:
        scratch += [
            pltpu.VMEM_SHARED((N_PAD,), jnp.float32),
            pltpu.VMEM((1024,), jnp.float32),
            pltpu.VMEM((1024,), jnp.float32),
            pltpu.SemaphoreType.DMA,
        ]

    def body(*refs):
        tables = refs[:n_tables]
        src_hbm, dst_hbm = refs[n_tables], refs[n_tables + 1]
        refs = refs[n_tables + 2:]
        if do_deg:
            out_hbm, degp_hbm = refs[0], refs[1]
            (acc, sidx0, sidx1, didx0, didx1, rows0, rows1,
             sg0, sg1, ss0, ss1, deg_acc, ones_v, zflat_v, sem_d) = refs[2:]
        else:
            out_hbm = refs[0]
            (acc, sidx0, sidx1, didx0, didx1, rows0, rows1,
             sg0, sg1, ss0, ss1) = refs[1:]
        sidx, didx = (sidx0, sidx1), (didx0, didx1)
        rows, sem_g, sem_s = (rows0, rows1), (sg0, sg1), (ss0, ss1)

        c = lax.axis_index("c")
        s = lax.axis_index("s")
        base_r = s * R
        base_t = s * (E // NS)
        nfull, rem = R // k, R % k

        if do_deg:
            _fill_1d(zflat_v, 1024, 0.0)
            _fill_1d(ones_v, 1024, 1.0)

        def load_idx(b, j):
            pltpu.sync_copy(src_hbm.at[pl.ds(base_t + j * k, k)], sidx[b])
            pltpu.sync_copy(dst_hbm.at[pl.ds(base_t + j * k, k)], didx[b])

        def pipeline(tbl, dodeg, ci):
            for b in range(2):
                load_idx(b, b)
                pltpu.async_copy(tbl.at[sidx[b]], rows[b], sem_g[b])

            @pl.loop(0, n, step=2)
            def _(i):
                for b in range(2):
                    j = i + b
                    pltpu.make_async_copy(tbl.at[pl.ds(0, k)], rows[b],
                                          sem_g[b]).wait()
                    sd = pltpu.async_copy(rows[b], acc.at[didx[b]],
                                          sem_s[b], add=True)
                    if dodeg:
                        cond = s < NS // NC if ci == 0 else s >= NS // NC

                        @pl.when(cond)
                        def _():
                            pltpu.async_copy(ones_v.at[pl.ds(0, k)],
                                             deg_acc.at[didx[b]],
                                             sem_d, add=True).wait()
                    sd.wait()

                    @pl.when(j + 2 < n)
                    def _():
                        load_idx(b, j + 2)
                        pltpu.async_copy(tbl.at[sidx[b]], rows[b], sem_g[b])

        for q in range(passes):
            def zb(i, _):
                rows0[i, pl.ds(0, 16)] = jnp.zeros((16,), jnp.float32)
                return 0
            lax.fori_loop(0, k, zb, 0)
            for j2 in range(nfull):
                pltpu.sync_copy(rows0, acc.at[pl.ds(base_r + j2 * k, k)])
            if rem:
                pltpu.sync_copy(rows0.at[pl.ds(0, rem)],
                                acc.at[pl.ds(base_r + nfull * k, rem)])
            dodeg = do_deg and q == 0
            if dodeg:
                for j2 in range(3):
                    pltpu.sync_copy(zflat_v,
                                    deg_acc.at[pl.ds(base_r + j2 * 1024, 1024)])
                pltpu.sync_copy(zflat_v.at[pl.ds(0, R - 3 * 1024)],
                                deg_acc.at[pl.ds(base_r + 3 * 1024,
                                                 R - 3 * 1024)])
            plsc.subcore_barrier()

            @pl.when(c == 0)
            def _():
                pipeline(tables[q], dodeg, 0)

            @pl.when(c == 1)
            def _():
                pipeline(tables[passes + q], dodeg, 1)

            plsc.subcore_barrier()

            out_base = (c * passes + q) * N_PAD + base_r
            pltpu.sync_copy(acc.at[pl.ds(base_r, R)],
                            out_hbm.at[pl.ds(out_base, R)])
            if dodeg:
                pltpu.sync_copy(deg_acc.at[pl.ds(base_r, R)],
                                degp_hbm.at[pl.ds(c * N_PAD + base_r, R)])

    return pl.kernel(body, out_type=tuple(out_types), mesh=_get_mesh(),
                     scratch_types=scratch,
                     compiler_params=pltpu.CompilerParams(
                         use_tc_tiling_on_sc=False))


_scatter_cache = {}


def _get_scatter(n_tables: int, k: int, do_deg: bool):
    key = (n_tables, k, do_deg)
    if key not in _scatter_cache:
        _scatter_cache[key] = _make_scatter(n_tables, k, do_deg)
    return _scatter_cache[key]


B3 = B * 3
_TG = B3 // (NC * NS)


def _team_gather_body(emb_hbm, tidx_hbm, trows_hbm, idx_v, rows_v, sem):
    c = lax.axis_index("c")
    s = lax.axis_index("s")
    w = s * NC + c
    base = w * _TG
    pltpu.sync_copy(tidx_hbm.at[pl.ds(base, _TG)], idx_v)
    pltpu.async_copy(emb_hbm.at[idx_v], rows_v, sem).wait()
    pltpu.sync_copy(rows_v, trows_hbm.at[pl.ds(base, _TG)])


@functools.cache
def _get_team_gather():
    return pl.kernel(
        _team_gather_body,
        out_type=jax.ShapeDtypeStruct((B3, DE), jnp.float32),
        mesh=_get_mesh(),
        scratch_types=[
            pltpu.VMEM((_TG,), jnp.int32),
            pltpu.VMEM((_TG, DE), jnp.float32),
            pltpu.SemaphoreType.DMA,
        ],
        compiler_params=pltpu.CompilerParams(use_tc_tiling_on_sc=False),
    )


BN = 1024
GRID_N = -(-N // BN)


def _prep_body(x_ref, w1lt_ref, w1rt_ref, b1_ref,
               t0_ref, t1_ref, t2_ref, t3_ref, q_ref):
    xb = x_ref[...]
    t = jnp.dot(xb, w1lt_ref[...], preferred_element_type=jnp.float32)
    t0_ref[...] = t[:, 0 * DE2:1 * DE2]
    t1_ref[...] = t[:, 1 * DE2:2 * DE2]
    t2_ref[...] = t[:, 2 * DE2:3 * DE2]
    t3_ref[...] = t[:, 3 * DE2:4 * DE2]
    q_ref[...] = jnp.dot(xb, w1rt_ref[...],
                         preferred_element_type=jnp.float32) + b1_ref[...]


def _prep(x, w1lt, w1rt, b1row):
    return pl.pallas_call(
        _prep_body,
        grid=(GRID_N,),
        in_specs=[
            pl.BlockSpec((BN, DIN), lambda i: (i, 0)),
            pl.BlockSpec((DIN, DH), lambda i: (0, 0)),
            pl.BlockSpec((DIN, DH), lambda i: (0, 0)),
            pl.BlockSpec((1, DH), lambda i: (0, 0)),
        ],
        out_specs=[pl.BlockSpec((BN, DE2), lambda i: (i, 0))] * 4
        + [pl.BlockSpec((BN, DH), lambda i: (i, 0))],
        out_shape=[jax.ShapeDtypeStruct((N, DE2), jnp.float32)] * 4
        + [jax.ShapeDtypeStruct((N, DH), jnp.float32)],
    )(x, w1lt, w1rt, b1row)


def _mid_body(s1_ref, degp_ref, q_ref, w2lt_ref, w2rt_ref, b2_ref,
              m2a_ref, m2b_ref, r2_ref, recip_ref):
    deg = degp_ref[0] + degp_ref[1]
    rc = 1.0 / jnp.maximum(deg, 1.0)
    mean = jnp.concatenate([s1_ref[0], s1_ref[1], s1_ref[2], s1_ref[3]],
                           axis=1) * rc[:, None]
    h = jnp.maximum(mean + q_ref[...], 0.0)
    m2 = jnp.dot(h, w2lt_ref[...], preferred_element_type=jnp.float32)
    m2a_ref[...] = m2[:, :DE2]
    m2b_ref[...] = m2[:, DE2:]
    r2_ref[...] = jnp.dot(h, w2rt_ref[...],
                          preferred_element_type=jnp.float32) + b2_ref[...]
    recip_ref[...] = rc


def _mid(s1, degp, q, w2lt, w2rt, b2row):
    return pl.pallas_call(
        _mid_body,
        grid=(GRID_N,),
        in_specs=[
            pl.BlockSpec((2 * NC, BN, DE2), lambda i: (0, i, 0)),
            pl.BlockSpec((NC, BN), lambda i: (0, i)),
            pl.BlockSpec((BN, DH), lambda i: (i, 0)),
            pl.BlockSpec((DH, DE), lambda i: (0, 0)),
            pl.BlockSpec((DH, DE), lambda i: (0, 0)),
            pl.BlockSpec((1, DE), lambda i: (0, 0)),
        ],
        out_specs=[
            pl.BlockSpec((BN, DE2), lambda i: (i, 0)),
            pl.BlockSpec((BN, DE2), lambda i: (i, 0)),
            pl.BlockSpec((BN, DE), lambda i: (i, 0)),
            pl.BlockSpec((BN,), lambda i: (i,)),
        ],
        out_shape=[
            jax.ShapeDtypeStruct((N, DE2), jnp.float32),
            jax.ShapeDtypeStruct((N, DE2), jnp.float32),
            jax.ShapeDtypeStruct((N, DE), jnp.float32),
            jax.ShapeDtypeStruct((N,), jnp.float32),
        ],
    )(s1, degp, q, w2lt, w2rt, b2row)


def _post_body(s2_ref, recip_ref, r2_ref, emb_ref):
    mean2 = jnp.concatenate([s2_ref[0], s2_ref[1]], axis=1)
    emb_ref[...] = mean2 * recip_ref[...][:, None] + r2_ref[...]


def _post(s2, recip, r2):
    return pl.pallas_call(
        _post_body,
        grid=(GRID_N,),
        in_specs=[
            pl.BlockSpec((NC, BN, DE2), lambda i: (0, i, 0)),
            pl.BlockSpec((BN,), lambda i: (i,)),
            pl.BlockSpec((BN, DE), lambda i: (i, 0)),
        ],
        out_specs=pl.BlockSpec((BN, DE), lambda i: (i, 0)),
        out_shape=jax.ShapeDtypeStruct((N, DE), jnp.float32),
    )(s2, recip, r2)


BD = 1024


def _dur_body(tvec_ref, wpt_ref, bp_ref, out_ref):
    out_ref[...] = jnp.dot(tvec_ref[...], wpt_ref[...],
                           preferred_element_type=jnp.float32) + bp_ref[...]


def _dur(tvec, wpt, bp11):
    return pl.pallas_call(
        _dur_body,
        grid=(B // BD,),
        in_specs=[
            pl.BlockSpec((BD, 3 * DE), lambda i: (i, 0)),
            pl.BlockSpec((3 * DE, 1), lambda i: (0, 0)),
            pl.BlockSpec((1, 1), lambda i: (0, 0)),
        ],
        out_specs=pl.BlockSpec((BD, 1), lambda i: (i, 0)),
        out_shape=jax.ShapeDtypeStruct((B, 1), jnp.float32),
    )(tvec, wpt, bp11)


def kernel(x, edge_index, team_indices, W1l, b1, W1r, W2l, b2, W2r, Wp, bp):
    src = edge_index[0]
    dst = edge_index[1]

    t0, t1, t2, t3, q = _prep(x, W1l.T, W1r.T, b1.reshape(1, DH))
    s1_flat, degp_flat = _get_scatter(4, K2, True)(t0, t1, t2, t3, src, dst)
    s1 = s1_flat.reshape(2 * NC, N_PAD, DE2)
    degp = degp_flat.reshape(NC, N_PAD)

    m2a, m2b, r2, recip = _mid(s1, degp, q, W2l.T, W2r.T, b2.reshape(1, DE))
    (s2_flat,) = _get_scatter(2, K2, False)(m2a, m2b, src, dst)
    s2 = s2_flat.reshape(NC, N_PAD, DE2)

    emb = _post(s2, recip, r2)

    trows = _get_team_gather()(emb, team_indices.reshape(B3))
    dur = _dur(trows.reshape(B, 3 * DE), Wp.T, bp.reshape(1, 1))
    return emb, dur[:, 0]


# speedup vs baseline: 10.6774x; 1.0485x over previous
import functools

import jax
import jax.numpy as jnp
from jax import lax
from jax.experimental import pallas as pl
from jax.experimental.pallas import tpu as pltpu
from jax.experimental.pallas import tpu_sc as plsc

N = 50000
E = 800000
DIN = 64
DH = 64
DE = 32
B = 16384

NC = 2
NS = 16
R = 3128
N_PAD = NS * R
K2 = 1000
DE2 = DE // 2


@functools.cache
def _get_mesh():
    return plsc.VectorSubcoreMesh(core_axis_name="c", subcore_axis_name="s",
                                  num_cores=NC, num_subcores=NS)


def _fill_1d(ref, n, value):
    def body(i, _):
        ref[pl.ds(i * 16, 16)] = jnp.full((16,), value, jnp.float32)
        return 0
    lax.fori_loop(0, n // 16, body, 0)


def _make_scatter(n_tables: int, k: int, do_deg: bool):
    passes = n_tables // NC
    n = E // NS // k
    W = 16

    out_types = [jax.ShapeDtypeStruct((n_tables * N_PAD, W), jnp.float32)]
    if do_deg:
        out_types.append(jax.ShapeDtypeStruct((NC * N_PAD,), jnp.float32))

    scratch = [
        pltpu.VMEM_SHARED((N_PAD, W), jnp.float32),
        pltpu.VMEM((k,), jnp.int32), pltpu.VMEM((k,), jnp.int32),
        pltpu.VMEM((k,), jnp.int32), pltpu.VMEM((k,), jnp.int32),
        pltpu.VMEM((k, W), jnp.float32), pltpu.VMEM((k, W), jnp.float32),
        pltpu.SemaphoreType.DMA, pltpu.SemaphoreType.DMA,
        pltpu.SemaphoreType.DMA, pltpu.SemaphoreType.DMA,
    ]
    if do_deg:
        scratch += [
            pltpu.VMEM_SHARED((N_PAD,), jnp.float32),
            pltpu.VMEM((1024,), jnp.float32),
            pltpu.VMEM((1024,), jnp.float32),
            pltpu.SemaphoreType.DMA,
        ]

    def body(*refs):
        tables = refs[:n_tables]
        src_hbm, dst_hbm = refs[n_tables], refs[n_tables + 1]
        refs = refs[n_tables + 2:]
        if do_deg:
            out_hbm, degp_hbm = refs[0], refs[1]
            (acc, sidx0, sidx1, didx0, didx1, rows0, rows1,
             sg0, sg1, ss0, ss1, deg_acc, ones_v, zflat_v, sem_d) = refs[2:]
        else:
            out_hbm = refs[0]
            (acc, sidx0, sidx1, didx0, didx1, rows0, rows1,
             sg0, sg1, ss0, ss1) = refs[1:]
        sidx, didx = (sidx0, sidx1), (didx0, didx1)
        rows, sem_g, sem_s = (rows0, rows1), (sg0, sg1), (ss0, ss1)

        c = lax.axis_index("c")
        s = lax.axis_index("s")
        base_r = s * R
        base_t = s * (E // NS)
        nfull, rem = R // k, R % k

        if do_deg:
            _fill_1d(zflat_v, 1024, 0.0)
            _fill_1d(ones_v, 1024, 1.0)

        def pipeline(tbl, dodeg, ci):
            for b in range(2):
                pltpu.sync_copy(src_hbm.at[pl.ds(base_t + b * k, k)], sidx[b])
                pltpu.sync_copy(dst_hbm.at[pl.ds(base_t + b * k, k)], didx[b])
                pltpu.async_copy(tbl.at[sidx[b]], rows[b], sem_g[b])

            @pl.loop(0, n, step=2)
            def _(i):
                for b in range(2):
                    j = i + b
                    pltpu.make_async_copy(tbl.at[pl.ds(0, k)], rows[b],
                                          sem_g[b]).wait()
                    sd = pltpu.async_copy(rows[b], acc.at[didx[b]],
                                          sem_s[b], add=True)
                    if dodeg:
                        cond = s < NS // NC if ci == 0 else s >= NS // NC

                        @pl.when(cond)
                        def _():
                            pltpu.async_copy(ones_v.at[pl.ds(0, k)],
                                             deg_acc.at[didx[b]],
                                             sem_d, add=True).wait()

                    @pl.when(j + 2 < n)
                    def _():
                        pltpu.sync_copy(src_hbm.at[pl.ds(base_t + (j + 2) * k,
                                                         k)], sidx[b])
                    sd.wait()

                    @pl.when(j + 2 < n)
                    def _():
                        pltpu.sync_copy(dst_hbm.at[pl.ds(base_t + (j + 2) * k,
                                                         k)], didx[b])
                        pltpu.async_copy(tbl.at[sidx[b]], rows[b], sem_g[b])

        for q in range(passes):
            def zb(i, _):
                rows0[i, pl.ds(0, 16)] = jnp.zeros((16,), jnp.float32)
                return 0
            lax.fori_loop(0, k, zb, 0)
            for j2 in range(nfull):
                pltpu.sync_copy(rows0, acc.at[pl.ds(base_r + j2 * k, k)])
            if rem:
                pltpu.sync_copy(rows0.at[pl.ds(0, rem)],
                                acc.at[pl.ds(base_r + nfull * k, rem)])
            dodeg = do_deg and q == 0
            if dodeg:
                for j2 in range(3):
                    pltpu.sync_copy(zflat_v,
                                    deg_acc.at[pl.ds(base_r + j2 * 1024, 1024)])
                pltpu.sync_copy(zflat_v.at[pl.ds(0, R - 3 * 1024)],
                                deg_acc.at[pl.ds(base_r + 3 * 1024,
                                                 R - 3 * 1024)])
            plsc.subcore_barrier()

            @pl.when(c == 0)
            def _():
                pipeline(tables[q], dodeg, 0)

            @pl.when(c == 1)
            def _():
                pipeline(tables[passes + q], dodeg, 1)

            plsc.subcore_barrier()

            out_base = (c * passes + q) * N_PAD + base_r
            pltpu.sync_copy(acc.at[pl.ds(base_r, R)],
                            out_hbm.at[pl.ds(out_base, R)])
            if dodeg:
                pltpu.sync_copy(deg_acc.at[pl.ds(base_r, R)],
                                degp_hbm.at[pl.ds(c * N_PAD + base_r, R)])

    return pl.kernel(body, out_type=tuple(out_types), mesh=_get_mesh(),
                     scratch_types=scratch,
                     compiler_params=pltpu.CompilerParams(
                         use_tc_tiling_on_sc=False))


_scatter_cache = {}


def _get_scatter(n_tables: int, k: int, do_deg: bool):
    key = (n_tables, k, do_deg)
    if key not in _scatter_cache:
        _scatter_cache[key] = _make_scatter(n_tables, k, do_deg)
    return _scatter_cache[key]


B3 = B * 3
_TG = B3 // (NC * NS)


def _team_gather_body(emb_hbm, tidx_hbm, trows_hbm, idx_v, rows_v, sem):
    c = lax.axis_index("c")
    s = lax.axis_index("s")
    w = s * NC + c
    base = w * _TG
    pltpu.sync_copy(tidx_hbm.at[pl.ds(base, _TG)], idx_v)
    pltpu.async_copy(emb_hbm.at[idx_v], rows_v, sem).wait()
    pltpu.sync_copy(rows_v, trows_hbm.at[pl.ds(base, _TG)])


@functools.cache
def _get_team_gather():
    return pl.kernel(
        _team_gather_body,
        out_type=jax.ShapeDtypeStruct((B3, DE), jnp.float32),
        mesh=_get_mesh(),
        scratch_types=[
            pltpu.VMEM((_TG,), jnp.int32),
            pltpu.VMEM((_TG, DE), jnp.float32),
            pltpu.SemaphoreType.DMA,
        ],
        compiler_params=pltpu.CompilerParams(use_tc_tiling_on_sc=False),
    )


BN = 1024
GRID_N = -(-N // BN)


def _prep_body(x_ref, w1lt_ref, w1rt_ref, b1_ref,
               t0_ref, t1_ref, t2_ref, t3_ref, q_ref):
    xb = x_ref[...]
    t = jnp.dot(xb, w1lt_ref[...], preferred_element_type=jnp.float32)
    t0_ref[...] = t[:, 0 * DE2:1 * DE2]
    t1_ref[...] = t[:, 1 * DE2:2 * DE2]
    t2_ref[...] = t[:, 2 * DE2:3 * DE2]
    t3_ref[...] = t[:, 3 * DE2:4 * DE2]
    q_ref[...] = jnp.dot(xb, w1rt_ref[...],
                         preferred_element_type=jnp.float32) + b1_ref[...]


def _prep(x, w1lt, w1rt, b1row):
    return pl.pallas_call(
        _prep_body,
        grid=(GRID_N,),
        in_specs=[
            pl.BlockSpec((BN, DIN), lambda i: (i, 0)),
            pl.BlockSpec((DIN, DH), lambda i: (0, 0)),
            pl.BlockSpec((DIN, DH), lambda i: (0, 0)),
            pl.BlockSpec((1, DH), lambda i: (0, 0)),
        ],
        out_specs=[pl.BlockSpec((BN, DE2), lambda i: (i, 0))] * 4
        + [pl.BlockSpec((BN, DH), lambda i: (i, 0))],
        out_shape=[jax.ShapeDtypeStruct((N, DE2), jnp.float32)] * 4
        + [jax.ShapeDtypeStruct((N, DH), jnp.float32)],
    )(x, w1lt, w1rt, b1row)


def _mid_body(s1_ref, degp_ref, q_ref, w2lt_ref, w2rt_ref, b2_ref,
              m2a_ref, m2b_ref, r2_ref, recip_ref):
    deg = degp_ref[0] + degp_ref[1]
    rc = 1.0 / jnp.maximum(deg, 1.0)
    mean = jnp.concatenate([s1_ref[0], s1_ref[1], s1_ref[2], s1_ref[3]],
                           axis=1) * rc[:, None]
    h = jnp.maximum(mean + q_ref[...], 0.0)
    m2 = jnp.dot(h, w2lt_ref[...], preferred_element_type=jnp.float32)
    m2a_ref[...] = m2[:, :DE2]
    m2b_ref[...] = m2[:, DE2:]
    r2_ref[...] = jnp.dot(h, w2rt_ref[...],
                          preferred_element_type=jnp.float32) + b2_ref[...]
    recip_ref[...] = rc


def _mid(s1, degp, q, w2lt, w2rt, b2row):
    return pl.pallas_call(
        _mid_body,
        grid=(GRID_N,),
        in_specs=[
            pl.BlockSpec((2 * NC, BN, DE2), lambda i: (0, i, 0)),
            pl.BlockSpec((NC, BN), lambda i: (0, i)),
            pl.BlockSpec((BN, DH), lambda i: (i, 0)),
            pl.BlockSpec((DH, DE), lambda i: (0, 0)),
            pl.BlockSpec((DH, DE), lambda i: (0, 0)),
            pl.BlockSpec((1, DE), lambda i: (0, 0)),
        ],
        out_specs=[
            pl.BlockSpec((BN, DE2), lambda i: (i, 0)),
            pl.BlockSpec((BN, DE2), lambda i: (i, 0)),
            pl.BlockSpec((BN, DE), lambda i: (i, 0)),
            pl.BlockSpec((BN,), lambda i: (i,)),
        ],
        out_shape=[
            jax.ShapeDtypeStruct((N, DE2), jnp.float32),
            jax.ShapeDtypeStruct((N, DE2), jnp.float32),
            jax.ShapeDtypeStruct((N, DE), jnp.float32),
            jax.ShapeDtypeStruct((N,), jnp.float32),
        ],
    )(s1, degp, q, w2lt, w2rt, b2row)


def _post_body(s2_ref, recip_ref, r2_ref, emb_ref):
    mean2 = jnp.concatenate([s2_ref[0], s2_ref[1]], axis=1)
    emb_ref[...] = mean2 * recip_ref[...][:, None] + r2_ref[...]


def _post(s2, recip, r2):
    return pl.pallas_call(
        _post_body,
        grid=(GRID_N,),
        in_specs=[
            pl.BlockSpec((NC, BN, DE2), lambda i: (0, i, 0)),
            pl.BlockSpec((BN,), lambda i: (i,)),
            pl.BlockSpec((BN, DE), lambda i: (i, 0)),
        ],
        out_specs=pl.BlockSpec((BN, DE), lambda i: (i, 0)),
        out_shape=jax.ShapeDtypeStruct((N, DE), jnp.float32),
    )(s2, recip, r2)


BD = 1024


def _dur_body(tvec_ref, wpt_ref, bp_ref, out_ref):
    out_ref[...] = jnp.dot(tvec_ref[...], wpt_ref[...],
                           preferred_element_type=jnp.float32) + bp_ref[...]


def _dur(tvec, wpt, bp11):
    return pl.pallas_call(
        _dur_body,
        grid=(B // BD,),
        in_specs=[
            pl.BlockSpec((BD, 3 * DE), lambda i: (i, 0)),
            pl.BlockSpec((3 * DE, 1), lambda i: (0, 0)),
            pl.BlockSpec((1, 1), lambda i: (0, 0)),
        ],
        out_specs=pl.BlockSpec((BD, 1), lambda i: (i, 0)),
        out_shape=jax.ShapeDtypeStruct((B, 1), jnp.float32),
    )(tvec, wpt, bp11)


def kernel(x, edge_index, team_indices, W1l, b1, W1r, W2l, b2, W2r, Wp, bp):
    src = edge_index[0]
    dst = edge_index[1]

    t0, t1, t2, t3, q = _prep(x, W1l.T, W1r.T, b1.reshape(1, DH))
    s1_flat, degp_flat = _get_scatter(4, K2, True)(t0, t1, t2, t3, src, dst)
    s1 = s1_flat.reshape(2 * NC, N_PAD, DE2)
    degp = degp_flat.reshape(NC, N_PAD)

    m2a, m2b, r2, recip = _mid(s1, degp, q, W2l.T, W2r.T, b2.reshape(1, DE))
    (s2_flat,) = _get_scatter(2, K2, False)(m2a, m2b, src, dst)
    s2 = s2_flat.reshape(NC, N_PAD, DE2)

    emb = _post(s2, recip, r2)

    trows = _get_team_gather()(emb, team_indices.reshape(B3))
    dur = _dur(trows.reshape(B, 3 * DE), Wp.T, bp.reshape(1, 1))
    return emb, dur[:, 0]


# speedup vs baseline: 11.0906x; 1.0387x over previous
import functools

import jax
import jax.numpy as jnp
from jax import lax
from jax.experimental import pallas as pl
from jax.experimental.pallas import tpu as pltpu
from jax.experimental.pallas import tpu_sc as plsc

N = 50000
E = 800000
DIN = 64
DH = 64
DE = 32
B = 16384

NC = 2
NS = 16
R = 3128
N_PAD = NS * R
K2 = 1000
DE2 = DE // 2


@functools.cache
def _get_mesh():
    return plsc.VectorSubcoreMesh(core_axis_name="c", subcore_axis_name="s",
                                  num_cores=NC, num_subcores=NS)


def _fill_1d(ref, n, value):
    def body(i, _):
        ref[pl.ds(i * 16, 16)] = jnp.full((16,), value, jnp.float32)
        return 0
    lax.fori_loop(0, n // 16, body, 0)


def _make_scatter(n_tables: int, k: int, do_deg: bool):
    passes = n_tables // NC
    n = E // NS // k
    W = 16

    out_types = [jax.ShapeDtypeStruct((n_tables * N_PAD, W), jnp.float32)]
    if do_deg:
        out_types.append(jax.ShapeDtypeStruct((NC * N_PAD,), jnp.float32))

    scratch = [
        pltpu.VMEM_SHARED((N_PAD, W), jnp.float32),
        pltpu.VMEM((k,), jnp.int32), pltpu.VMEM((k,), jnp.int32),
        pltpu.VMEM((k,), jnp.int32), pltpu.VMEM((k,), jnp.int32),
        pltpu.VMEM((k, W), jnp.float32), pltpu.VMEM((k, W), jnp.float32),
        pltpu.SemaphoreType.DMA, pltpu.SemaphoreType.DMA,
        pltpu.SemaphoreType.DMA, pltpu.SemaphoreType.DMA,
    ]
    if do_deg:
        scratch += [
            pltpu.VMEM_SHARED((N_PAD,), jnp.float32),
            pltpu.VMEM((1024,), jnp.float32),
            pltpu.VMEM((1024,), jnp.float32),
            pltpu.SemaphoreType.DMA,
        ]

    def body(*refs):
        tables = refs[:n_tables]
        src_hbm, dst_hbm = refs[n_tables], refs[n_tables + 1]
        refs = refs[n_tables + 2:]
        if do_deg:
            out_hbm, degp_hbm = refs[0], refs[1]
            (acc, sidx0, sidx1, didx0, didx1, rows0, rows1,
             sg0, sg1, ss0, ss1, deg_acc, ones_v, zflat_v, sem_d) = refs[2:]
        else:
            out_hbm = refs[0]
            (acc, sidx0, sidx1, didx0, didx1, rows0, rows1,
             sg0, sg1, ss0, ss1) = refs[1:]
        sidx, didx = (sidx0, sidx1), (didx0, didx1)
        rows, sem_g, sem_s = (rows0, rows1), (sg0, sg1), (ss0, ss1)

        c = lax.axis_index("c")
        s = lax.axis_index("s")
        base_r = s * R
        base_t = s * (E // NS)
        nfull, rem = R // k, R % k

        if do_deg:
            _fill_1d(zflat_v, 1024, 0.0)
            _fill_1d(ones_v, 1024, 1.0)

        def pipeline(tbl, dodeg, ci):
            for b in range(2):
                pltpu.sync_copy(src_hbm.at[pl.ds(base_t + b * k, k)], sidx[b])
                pltpu.sync_copy(dst_hbm.at[pl.ds(base_t + b * k, k)], didx[b])
                pltpu.async_copy(tbl.at[sidx[b]], rows[b], sem_g[b])

            @pl.loop(0, n, step=2)
            def _(i):
                for b in range(2):
                    j = i + b
                    pltpu.make_async_copy(tbl.at[pl.ds(0, k)], rows[b],
                                          sem_g[b]).wait()
                    sd = pltpu.async_copy(rows[b], acc.at[didx[b]],
                                          sem_s[b], add=True)
                    if dodeg:
                        cond = s < NS // NC if ci == 0 else s >= NS // NC

                        @pl.when(cond)
                        def _():
                            pltpu.async_copy(ones_v.at[pl.ds(0, k)],
                                             deg_acc.at[didx[b]],
                                             sem_d, add=True).wait()

                    @pl.when(j + 2 < n)
                    def _():
                        pltpu.sync_copy(src_hbm.at[pl.ds(base_t + (j + 2) * k,
                                                         k)], sidx[b])
                    sd.wait()

                    @pl.when(j + 2 < n)
                    def _():
                        pltpu.sync_copy(dst_hbm.at[pl.ds(base_t + (j + 2) * k,
                                                         k)], didx[b])
                        pltpu.async_copy(tbl.at[sidx[b]], rows[b], sem_g[b])

        for q in range(passes):
            def zb(i, _):
                rows0[i, pl.ds(0, 16)] = jnp.zeros((16,), jnp.float32)
                return 0
            lax.fori_loop(0, k, zb, 0)
            for j2 in range(nfull):
                pltpu.sync_copy(rows0, acc.at[pl.ds(base_r + j2 * k, k)])
            if rem:
                pltpu.sync_copy(rows0.at[pl.ds(0, rem)],
                                acc.at[pl.ds(base_r + nfull * k, rem)])
            dodeg = do_deg and q == 0
            if dodeg:
                for j2 in range(3):
                    pltpu.sync_copy(zflat_v,
                                    deg_acc.at[pl.ds(base_r + j2 * 1024, 1024)])
                pltpu.sync_copy(zflat_v.at[pl.ds(0, R - 3 * 1024)],
                                deg_acc.at[pl.ds(base_r + 3 * 1024,
                                                 R - 3 * 1024)])
            plsc.subcore_barrier()

            @pl.when(c == 0)
            def _():
                pipeline(tables[q], dodeg, 0)

            @pl.when(c == 1)
            def _():
                pipeline(tables[passes + q], dodeg, 1)

            plsc.subcore_barrier()

            out_base = (c * passes + q) * N_PAD + base_r
            pltpu.sync_copy(acc.at[pl.ds(base_r, R)],
                            out_hbm.at[pl.ds(out_base, R)])
            if dodeg:
                pltpu.sync_copy(deg_acc.at[pl.ds(base_r, R)],
                                degp_hbm.at[pl.ds(c * N_PAD + base_r, R)])

    return pl.kernel(body, out_type=tuple(out_types), mesh=_get_mesh(),
                     scratch_types=scratch,
                     compiler_params=pltpu.CompilerParams(
                         use_tc_tiling_on_sc=False))


_scatter_cache = {}


def _get_scatter(n_tables: int, k: int, do_deg: bool):
    key = (n_tables, k, do_deg)
    if key not in _scatter_cache:
        _scatter_cache[key] = _make_scatter(n_tables, k, do_deg)
    return _scatter_cache[key]


B3 = B * 3
_TG = B3 // (NC * NS)
_TB = B // (NC * NS)


def _team_body(emb_hbm, tidx_hbm, wp_hbm, bp_hbm, dur_hbm,
               idx_v, rows_v, wp_v, bp_v, dur_v, sem):
    c = lax.axis_index("c")
    s = lax.axis_index("s")
    w = s * NC + c
    base = w * _TG
    pltpu.sync_copy(wp_hbm.at[pl.ds(0, 3 * DE)], wp_v)
    pltpu.sync_copy(bp_hbm.at[pl.ds(0, 16)], bp_v)
    pltpu.sync_copy(tidx_hbm.at[pl.ds(base, _TG)], idx_v)
    pltpu.async_copy(emb_hbm.at[idx_v], rows_v, sem).wait()

    wvs = [wp_v[pl.ds(seg * 16, 16)] for seg in range(6)]
    bpv = bp_v[pl.ds(0, 16)]
    lane = lax.iota(jnp.int32, 16)

    def group(g, _):
        dvec = jnp.zeros((16,), jnp.float32)
        for ln in range(16):
            t = g * 16 + ln
            acc = jnp.zeros((16,), jnp.float32)
            for r in range(3):
                for hh in range(2):
                    v = rows_v[3 * t + r, pl.ds(hh * 16, 16)]
                    acc = acc + v * wvs[r * 2 + hh]
            dvec = jnp.where(lane == ln, jnp.sum(acc), dvec)
        dur_v[pl.ds(g * 16, 16)] = dvec + bpv
        return 0
    lax.fori_loop(0, _TB // 16, group, 0)
    pltpu.sync_copy(dur_v, dur_hbm.at[pl.ds(w * _TB, _TB)])


@functools.cache
def _get_team():
    return pl.kernel(
        _team_body,
        out_type=jax.ShapeDtypeStruct((B,), jnp.float32),
        mesh=_get_mesh(),
        scratch_types=[
            pltpu.VMEM((_TG,), jnp.int32),
            pltpu.VMEM((_TG, DE), jnp.float32),
            pltpu.VMEM((3 * DE,), jnp.float32),
            pltpu.VMEM((16,), jnp.float32),
            pltpu.VMEM((_TB,), jnp.float32),
            pltpu.SemaphoreType.DMA,
        ],
        compiler_params=pltpu.CompilerParams(use_tc_tiling_on_sc=False,
                                             needs_layout_passes=False),
    )


BN = 1024
GRID_N = -(-N // BN)


def _prep_body(x_ref, w1lt_ref, w1rt_ref, b1_ref,
               t0_ref, t1_ref, t2_ref, t3_ref, q_ref):
    xb = x_ref[...]
    t = jnp.dot(xb, w1lt_ref[...], preferred_element_type=jnp.float32)
    t0_ref[...] = t[:, 0 * DE2:1 * DE2]
    t1_ref[...] = t[:, 1 * DE2:2 * DE2]
    t2_ref[...] = t[:, 2 * DE2:3 * DE2]
    t3_ref[...] = t[:, 3 * DE2:4 * DE2]
    q_ref[...] = jnp.dot(xb, w1rt_ref[...],
                         preferred_element_type=jnp.float32) + b1_ref[...]


def _prep(x, w1lt, w1rt, b1row):
    return pl.pallas_call(
        _prep_body,
        grid=(GRID_N,),
        in_specs=[
            pl.BlockSpec((BN, DIN), lambda i: (i, 0)),
            pl.BlockSpec((DIN, DH), lambda i: (0, 0)),
            pl.BlockSpec((DIN, DH), lambda i: (0, 0)),
            pl.BlockSpec((1, DH), lambda i: (0, 0)),
        ],
        out_specs=[pl.BlockSpec((BN, DE2), lambda i: (i, 0))] * 4
        + [pl.BlockSpec((BN, DH), lambda i: (i, 0))],
        out_shape=[jax.ShapeDtypeStruct((N, DE2), jnp.float32)] * 4
        + [jax.ShapeDtypeStruct((N, DH), jnp.float32)],
    )(x, w1lt, w1rt, b1row)


def _mid_body(s1_ref, degp_ref, q_ref, w2lt_ref, w2rt_ref, b2_ref,
              m2a_ref, m2b_ref, r2_ref, recip_ref):
    deg = degp_ref[0] + degp_ref[1]
    rc = 1.0 / jnp.maximum(deg, 1.0)
    mean = jnp.concatenate([s1_ref[0], s1_ref[1], s1_ref[2], s1_ref[3]],
                           axis=1) * rc[:, None]
    h = jnp.maximum(mean + q_ref[...], 0.0)
    m2 = jnp.dot(h, w2lt_ref[...], preferred_element_type=jnp.float32)
    m2a_ref[...] = m2[:, :DE2]
    m2b_ref[...] = m2[:, DE2:]
    r2_ref[...] = jnp.dot(h, w2rt_ref[...],
                          preferred_element_type=jnp.float32) + b2_ref[...]
    recip_ref[...] = rc


def _mid(s1, degp, q, w2lt, w2rt, b2row):
    return pl.pallas_call(
        _mid_body,
        grid=(GRID_N,),
        in_specs=[
            pl.BlockSpec((2 * NC, BN, DE2), lambda i: (0, i, 0)),
            pl.BlockSpec((NC, BN), lambda i: (0, i)),
            pl.BlockSpec((BN, DH), lambda i: (i, 0)),
            pl.BlockSpec((DH, DE), lambda i: (0, 0)),
            pl.BlockSpec((DH, DE), lambda i: (0, 0)),
            pl.BlockSpec((1, DE), lambda i: (0, 0)),
        ],
        out_specs=[
            pl.BlockSpec((BN, DE2), lambda i: (i, 0)),
            pl.BlockSpec((BN, DE2), lambda i: (i, 0)),
            pl.BlockSpec((BN, DE), lambda i: (i, 0)),
            pl.BlockSpec((BN,), lambda i: (i,)),
        ],
        out_shape=[
            jax.ShapeDtypeStruct((N, DE2), jnp.float32),
            jax.ShapeDtypeStruct((N, DE2), jnp.float32),
            jax.ShapeDtypeStruct((N, DE), jnp.float32),
            jax.ShapeDtypeStruct((N,), jnp.float32),
        ],
    )(s1, degp, q, w2lt, w2rt, b2row)


def _post_body(s2_ref, recip_ref, r2_ref, emb_ref):
    mean2 = jnp.concatenate([s2_ref[0], s2_ref[1]], axis=1)
    emb_ref[...] = mean2 * recip_ref[...][:, None] + r2_ref[...]


def _post(s2, recip, r2):
    return pl.pallas_call(
        _post_body,
        grid=(GRID_N,),
        in_specs=[
            pl.BlockSpec((NC, BN, DE2), lambda i: (0, i, 0)),
            pl.BlockSpec((BN,), lambda i: (i,)),
            pl.BlockSpec((BN, DE), lambda i: (i, 0)),
        ],
        out_specs=pl.BlockSpec((BN, DE), lambda i: (i, 0)),
        out_shape=jax.ShapeDtypeStruct((N, DE), jnp.float32),
    )(s2, recip, r2)


BD = 1024


def _dur_body(tvec_ref, wpt_ref, bp_ref, out_ref):
    out_ref[...] = jnp.dot(tvec_ref[...], wpt_ref[...],
                           preferred_element_type=jnp.float32) + bp_ref[...]


def _dur(tvec, wpt, bp11):
    return pl.pallas_call(
        _dur_body,
        grid=(B // BD,),
        in_specs=[
            pl.BlockSpec((BD, 3 * DE), lambda i: (i, 0)),
            pl.BlockSpec((3 * DE, 1), lambda i: (0, 0)),
            pl.BlockSpec((1, 1), lambda i: (0, 0)),
        ],
        out_specs=pl.BlockSpec((BD, 1), lambda i: (i, 0)),
        out_shape=jax.ShapeDtypeStruct((B, 1), jnp.float32),
    )(tvec, wpt, bp11)


def kernel(x, edge_index, team_indices, W1l, b1, W1r, W2l, b2, W2r, Wp, bp):
    src = edge_index[0]
    dst = edge_index[1]

    t0, t1, t2, t3, q = _prep(x, W1l.T, W1r.T, b1.reshape(1, DH))
    s1_flat, degp_flat = _get_scatter(4, K2, True)(t0, t1, t2, t3, src, dst)
    s1 = s1_flat.reshape(2 * NC, N_PAD, DE2)
    degp = degp_flat.reshape(NC, N_PAD)

    m2a, m2b, r2, recip = _mid(s1, degp, q, W2l.T, W2r.T, b2.reshape(1, DE))
    (s2_flat,) = _get_scatter(2, K2, False)(m2a, m2b, src, dst)
    s2 = s2_flat.reshape(NC, N_PAD, DE2)

    emb = _post(s2, recip, r2)

    dur = _get_team()(emb, team_indices.reshape(B3), Wp.reshape(3 * DE),
                      jnp.broadcast_to(bp, (16,)))
    return emb, dur
